# Initial kernel scaffold; baseline (speedup 1.0000x reference)
import jax
import jax.numpy as jnp
from jax.experimental import pallas as pl


def kernel(sparse_index_group_batch, sparse_offset_group_batch, weight):
    raise NotImplementedError("write your pallas kernel here")


# speedup vs baseline: 1.3529x; 1.3529x over previous
import functools

import jax
import jax.numpy as jnp
from jax import lax
from jax.experimental import pallas as pl
from jax.experimental.pallas import tpu as pltpu
from jax.experimental.pallas import tpu_sc as plsc

N_VOCAB = 100000
EMBED_DIM = 64
OFFSET = 20
BATCH = 4096

_INFO = plsc.get_sparse_core_info()
NC = _INFO.num_cores
NS = _INFO.num_subcores
NW = NC * NS
B_PER_W = BATCH // NW
NB = 32
NSUB = B_PER_W // NB
ROWS = NB * OFFSET


def _make_kernel():
    mesh = plsc.VectorSubcoreMesh(core_axis_name="c", subcore_axis_name="s")

    @functools.partial(
        pl.kernel,
        mesh=mesh,
        out_type=jax.ShapeDtypeStruct((BATCH, EMBED_DIM), jnp.float32),
        scratch_types=[
            pltpu.VMEM((ROWS,), jnp.int32),
            pltpu.VMEM((ROWS, EMBED_DIM), jnp.float32),
            pltpu.VMEM((NB, EMBED_DIM), jnp.float32),
            pltpu.SemaphoreType.DMA,
        ],
        compiler_params=pltpu.CompilerParams(use_tc_tiling_on_sc=False),
    )
    def embag(idx_hbm, table_hbm, out_hbm, idx_v, rows_v, out_v, sem):
        wid = lax.axis_index("s") * NC + lax.axis_index("c")
        for s in range(NSUB):
            base = wid * B_PER_W + s * NB
            pltpu.sync_copy(idx_hbm.at[pl.ds(base * OFFSET, ROWS)], idx_v)
            pltpu.async_copy(table_hbm.at[idx_v], rows_v, sem).wait()

            def body(b, _):
                for v in range(EMBED_DIM // 16):
                    sl = pl.ds(v * 16, 16)
                    acc = rows_v[b * OFFSET, sl]
                    for j in range(1, OFFSET):
                        acc = acc + rows_v[b * OFFSET + j, sl]
                    out_v[b, sl] = acc
                return 0

            lax.fori_loop(0, NB, body, 0)
            pltpu.sync_copy(out_v, out_hbm.at[pl.ds(base, NB)])

    return embag


_embag = _make_kernel()


@jax.jit
def kernel(sparse_index_group_batch, sparse_offset_group_batch, weight):
    del sparse_offset_group_batch
    idx = sparse_index_group_batch.astype(jnp.int32)
    return _embag(idx, weight)


# speedup vs baseline: 1.4452x; 1.0682x over previous
import functools

import jax
import jax.numpy as jnp
from jax import lax
from jax.experimental import pallas as pl
from jax.experimental.pallas import tpu as pltpu
from jax.experimental.pallas import tpu_sc as plsc

N_VOCAB = 100000
EMBED_DIM = 64
OFFSET = 20
BATCH = 4096

_INFO = plsc.get_sparse_core_info()
NC = _INFO.num_cores
NS = _INFO.num_subcores
NW = NC * NS
B_PER_W = BATCH // NW
NB = 32
NSUB = B_PER_W // NB
ROWS = NB * OFFSET


def _make_kernel():
    mesh = plsc.VectorSubcoreMesh(core_axis_name="c", subcore_axis_name="s")

    @functools.partial(
        pl.kernel,
        mesh=mesh,
        out_type=jax.ShapeDtypeStruct((BATCH, EMBED_DIM), jnp.float32),
        scratch_types=[
            pltpu.VMEM((B_PER_W * OFFSET,), jnp.int32),
            pltpu.VMEM((2, ROWS, EMBED_DIM), jnp.float32),
            pltpu.VMEM((2, NB, EMBED_DIM), jnp.float32),
            pltpu.SemaphoreType.DMA((2,)),
            pltpu.SemaphoreType.DMA((2,)),
        ],
        compiler_params=pltpu.CompilerParams(use_tc_tiling_on_sc=False),
    )
    def embag(idx_hbm, table_hbm, out_hbm, idx_v, rows_v, out_v, gsem, osem):
        wid = lax.axis_index("s") * NC + lax.axis_index("c")
        wbase = wid * B_PER_W
        pltpu.sync_copy(idx_hbm.at[pl.ds(wbase * OFFSET, B_PER_W * OFFSET)],
                        idx_v)

        def gather(s):
            return pltpu.async_copy(
                table_hbm.at[idx_v.at[pl.ds(s * ROWS, ROWS)]],
                rows_v.at[s % 2], gsem.at[s % 2])

        gc = {0: gather(0)}
        oc = {}
        for s in range(NSUB):
            if s + 1 < NSUB:
                gc[s + 1] = gather(s + 1)
            gc[s].wait()
            if s >= 2:
                oc[s - 2].wait()

            def body(b, _, buf=s % 2):
                for v in range(EMBED_DIM // 16):
                    sl = pl.ds(v * 16, 16)
                    acc = rows_v[buf, b * OFFSET, sl]
                    for j in range(1, OFFSET):
                        acc = acc + rows_v[buf, b * OFFSET + j, sl]
                    out_v[buf, b, sl] = acc
                return 0

            lax.fori_loop(0, NB, body, 0)
            oc[s] = pltpu.async_copy(
                out_v.at[s % 2],
                out_hbm.at[pl.ds(wbase + s * NB, NB)], osem.at[s % 2])
        oc[NSUB - 2].wait()
        oc[NSUB - 1].wait()

    return embag


_embag = _make_kernel()


@jax.jit
def kernel(sparse_index_group_batch, sparse_offset_group_batch, weight):
    del sparse_offset_group_batch
    idx = sparse_index_group_batch.astype(jnp.int32)
    return _embag(idx, weight)
